# baseline (speedup 1.0000x reference)
import jax
import jax.numpy as jnp
from jax.experimental import pallas as pl
from jax.experimental.pallas import tpu as pltpu

B2, L, H, D = 16, 2048, 16, 128
HD = H * D
BL = 256


def _copy_scatter_body(idx_ref, ck_ref, cv_ref, k_ref, v_ref, ok_ref, ov_ref):
    l = pl.program_id(1)
    ok_ref[...] = ck_ref[...]
    ov_ref[...] = cv_ref[...]
    r = idx_ref[0] - l * BL
    @pl.when(jnp.logical_and(r >= 0, r < BL))
    def _():
        ok_ref[0, pl.ds(r, 1), :] = k_ref[0]
        ov_ref[0, pl.ds(r, 1), :] = v_ref[0]


def kernel(cache_k, cache_v, k, v, current_idx):
    ck = cache_k.reshape(B2, L, HD)
    cv = cache_v.reshape(B2, L, HD)
    k3 = k.reshape(B2, 1, HD)
    v3 = v.reshape(B2, 1, HD)
    idx = jnp.asarray(current_idx, jnp.int32).reshape(1)

    grid_spec = pltpu.PrefetchScalarGridSpec(
        num_scalar_prefetch=1,
        grid=(B2, L // BL),
        in_specs=[
            pl.BlockSpec((1, BL, HD), lambda b, l, idx_ref: (b, l, 0)),
            pl.BlockSpec((1, BL, HD), lambda b, l, idx_ref: (b, l, 0)),
            pl.BlockSpec((1, 1, HD), lambda b, l, idx_ref: (b, 0, 0)),
            pl.BlockSpec((1, 1, HD), lambda b, l, idx_ref: (b, 0, 0)),
        ],
        out_specs=[
            pl.BlockSpec((1, BL, HD), lambda b, l, idx_ref: (b, l, 0)),
            pl.BlockSpec((1, BL, HD), lambda b, l, idx_ref: (b, l, 0)),
        ],
    )

    ok, ov = pl.pallas_call(
        _copy_scatter_body,
        grid_spec=grid_spec,
        out_shape=[
            jax.ShapeDtypeStruct((B2, L, HD), jnp.float32),
            jax.ShapeDtypeStruct((B2, L, HD), jnp.float32),
        ],
        compiler_params=pltpu.CompilerParams(
            dimension_semantics=("parallel", "parallel"),
        ),
    )(idx, ck, cv, k3, v3)
    return ok.reshape(B2, L, H, D), ov.reshape(B2, L, H, D)


# speedup vs baseline: 1.1555x; 1.1555x over previous
import jax
import jax.numpy as jnp
from jax.experimental import pallas as pl
from jax.experimental.pallas import tpu as pltpu

B2, L, H, D = 16, 2048, 16, 128
HD = H * D
CR = 512
NBUF = 6
NCH = (B2 * L) // CR


def _relay_body(idx_ref, ck, cv, k_ref, v_ref, ok, ov, buf, sem_r, sem_w):
    idx = idx_ref[0]
    chunks = []
    for i in range(NCH):
        chunks.append((ck, ok, k_ref, i))
        chunks.append((cv, ov, v_ref, i))

    def read(c, slot):
        src, _, _, i = c
        return pltpu.make_async_copy(
            src.at[pl.ds(i * CR, CR), :], buf.at[slot], sem_r.at[slot])

    def write(c, slot):
        _, dst, _, i = c
        return pltpu.make_async_copy(
            buf.at[slot], dst.at[pl.ds(i * CR, CR), :], sem_w.at[slot])

    for j in range(min(NBUF, len(chunks))):
        read(chunks[j], j).start()
    for j, c in enumerate(chunks):
        slot = j % NBUF
        read(c, slot).wait()
        _, _, new_ref, i = c
        b = (i * CR) // L
        r = (b * L + idx) - i * CR
        @pl.when(jnp.logical_and(r >= 0, r < CR))
        def _(slot=slot, new_ref=new_ref, b=b, r=r):
            buf[slot, pl.ds(r, 1), :] = new_ref[b]
        write(c, slot).start()
        nxt = j + NBUF
        if nxt < len(chunks):
            write(c, slot).wait()
            read(chunks[nxt], slot).start()
    for j in range(max(0, len(chunks) - NBUF), len(chunks)):
        write(chunks[j], j % NBUF).wait()


def kernel(cache_k, cache_v, k, v, current_idx):
    ck = cache_k.reshape(B2 * L, HD)
    cv = cache_v.reshape(B2 * L, HD)
    k3 = k.reshape(B2, 1, HD)
    v3 = v.reshape(B2, 1, HD)
    idx = jnp.asarray(current_idx, jnp.int32).reshape(1)

    ok, ov = pl.pallas_call(
        _relay_body,
        in_specs=[
            pl.BlockSpec(memory_space=pltpu.MemorySpace.SMEM),
            pl.BlockSpec(memory_space=pltpu.MemorySpace.HBM),
            pl.BlockSpec(memory_space=pltpu.MemorySpace.HBM),
            pl.BlockSpec(memory_space=pltpu.MemorySpace.VMEM),
            pl.BlockSpec(memory_space=pltpu.MemorySpace.VMEM),
        ],
        out_specs=[
            pl.BlockSpec(memory_space=pltpu.MemorySpace.HBM),
            pl.BlockSpec(memory_space=pltpu.MemorySpace.HBM),
        ],
        out_shape=[
            jax.ShapeDtypeStruct((B2 * L, HD), jnp.float32),
            jax.ShapeDtypeStruct((B2 * L, HD), jnp.float32),
        ],
        scratch_shapes=[
            pltpu.VMEM((NBUF, CR, HD), jnp.float32),
            pltpu.SemaphoreType.DMA((NBUF,)),
            pltpu.SemaphoreType.DMA((NBUF,)),
        ],
    )(idx, ck, cv, k3, v3)
    return ok.reshape(B2, L, H, D), ov.reshape(B2, L, H, D)


# speedup vs baseline: 1.1626x; 1.0062x over previous
import jax
import jax.numpy as jnp
from jax.experimental import pallas as pl
from jax.experimental.pallas import tpu as pltpu

B2, L, H, D = 16, 2048, 16, 128
HD = H * D
CR = 256
NBUF = 12
RA = 6
NCH = (B2 * L) // CR


def _relay_body(idx_ref, ck, cv, k_ref, v_ref, ok, ov, buf, sem_r, sem_w):
    idx = idx_ref[0]
    chunks = []
    for i in range(NCH):
        chunks.append((ck, ok, k_ref, i))
        chunks.append((cv, ov, v_ref, i))

    def read(c, slot):
        src, _, _, i = c
        return pltpu.make_async_copy(
            src.at[pl.ds(i * CR, CR), :], buf.at[slot], sem_r.at[slot])

    def write(c, slot):
        _, dst, _, i = c
        return pltpu.make_async_copy(
            buf.at[slot], dst.at[pl.ds(i * CR, CR), :], sem_w.at[slot])

    for j in range(min(RA, len(chunks))):
        read(chunks[j], j % NBUF).start()
    for j, c in enumerate(chunks):
        slot = j % NBUF
        read(c, slot).wait()
        _, _, new_ref, i = c
        b = (i * CR) // L
        r = (b * L + idx) - i * CR
        @pl.when(jnp.logical_and(r >= 0, r < CR))
        def _(slot=slot, new_ref=new_ref, b=b, r=r):
            buf[slot, pl.ds(r, 1), :] = new_ref[b]
        write(c, slot).start()
        nxt = j + RA
        if nxt < len(chunks):
            prev = nxt - NBUF
            if prev >= 0:
                write(chunks[prev], prev % NBUF).wait()
            read(chunks[nxt], nxt % NBUF).start()
    for j in range(max(0, len(chunks) - NBUF), len(chunks)):
        write(chunks[j], j % NBUF).wait()


def kernel(cache_k, cache_v, k, v, current_idx):
    ck = cache_k.reshape(B2 * L, HD)
    cv = cache_v.reshape(B2 * L, HD)
    k3 = k.reshape(B2, 1, HD)
    v3 = v.reshape(B2, 1, HD)
    idx = jnp.asarray(current_idx, jnp.int32).reshape(1)

    ok, ov = pl.pallas_call(
        _relay_body,
        in_specs=[
            pl.BlockSpec(memory_space=pltpu.MemorySpace.SMEM),
            pl.BlockSpec(memory_space=pltpu.MemorySpace.HBM),
            pl.BlockSpec(memory_space=pltpu.MemorySpace.HBM),
            pl.BlockSpec(memory_space=pltpu.MemorySpace.VMEM),
            pl.BlockSpec(memory_space=pltpu.MemorySpace.VMEM),
        ],
        out_specs=[
            pl.BlockSpec(memory_space=pltpu.MemorySpace.HBM),
            pl.BlockSpec(memory_space=pltpu.MemorySpace.HBM),
        ],
        out_shape=[
            jax.ShapeDtypeStruct((B2 * L, HD), jnp.float32),
            jax.ShapeDtypeStruct((B2 * L, HD), jnp.float32),
        ],
        scratch_shapes=[
            pltpu.VMEM((NBUF, CR, HD), jnp.float32),
            pltpu.SemaphoreType.DMA((NBUF,)),
            pltpu.SemaphoreType.DMA((NBUF,)),
        ],
    )(idx, ck, cv, k3, v3)
    return ok.reshape(B2, L, H, D), ov.reshape(B2, L, H, D)
